# baseline (speedup 1.0000x reference)
import functools

import jax
import jax.numpy as jnp
from jax.experimental import pallas as pl
from jax.experimental.pallas import tpu as pltpu

BR = 512
BC = 2048
LANES = 128


def _vq_kernel(x_ref, v_ref, x2_ref, v2_ref, col_ref, out_ref,
               m_sc, sv, sc, val_lo, idx_lo):
    j = pl.program_id(1)
    nsteps = pl.num_programs(1)
    nc = nsteps - 1
    half = nc // 2

    @pl.when(j > 0)
    def _process():
        roff = jax.lax.rem(j - 1, 2) * BR
        mp = m_sc[pl.ds(roff, BR), :]
        s = x2_ref[...] + v2_ref[...]
        d2 = jnp.float32(-2.0) * mp + s
        dist = jnp.sqrt(jnp.maximum(d2, 0.0))
        col = col_ref[...]

        @pl.when((j == 1) | (j == half + 1))
        def _reset():
            sv[...] = jnp.full((BR, LANES), jnp.inf, jnp.float32)
            sc[...] = jnp.zeros((BR, LANES), jnp.float32)

        tv = dist[:, :LANES]
        tc = jnp.broadcast_to(col[:, :LANES], (BR, LANES))
        for c in range(1, BC // LANES):
            ch = dist[:, c * LANES:(c + 1) * LANES]
            cc = col[:, c * LANES:(c + 1) * LANES]
            better = ch < tv
            tv = jnp.where(better, ch, tv)
            tc = jnp.where(better, cc, tc)
        bet = tv < sv[...]
        sv[...] = jnp.where(bet, tv, sv[...])
        sc[...] = jnp.where(bet, tc, sc[...])

        def _resolve():
            vst = sv[...]
            rv = jnp.min(vst, axis=1, keepdims=True)
            ri = jnp.min(jnp.where(vst == rv, sc[...], jnp.inf), axis=1,
                         keepdims=True)
            return rv, ri

        @pl.when(j == half)
        def _save_lo():
            rv, ri = _resolve()
            val_lo[...] = rv
            idx_lo[...] = ri

        @pl.when(j == nc)
        def _emit():
            rv, ri = _resolve()
            lo_rounded = val_lo[...].astype(jnp.bfloat16).astype(jnp.float32)
            take_hi = rv < lo_rounded
            best = jnp.where(take_hi, ri, idx_lo[...])
            out_ref[...] = best.astype(jnp.int32)

    @pl.when(j < nc)
    def _dot():
        woff = jax.lax.rem(j, 2) * BR
        m_sc[pl.ds(woff, BR), :] = jax.lax.dot_general(
            x_ref[...], v_ref[...],
            dimension_numbers=(((1,), (1,)), ((), ())),
            preferred_element_type=jnp.float32,
        )


@functools.partial(jax.jit, static_argnames=())
def kernel(x, vectors):
    shape = x.shape[:-1]
    d = x.shape[-1]
    xf = x.reshape(-1, d)
    n = xf.shape[0]
    k = vectors.shape[0]

    x2 = jnp.sum(xf * xf, axis=-1, keepdims=True)
    v2 = jnp.sum(vectors * vectors, axis=-1)[None, :]

    xb = xf.astype(jnp.bfloat16)
    vb = vectors.astype(jnp.bfloat16)
    colf = jnp.arange(k, dtype=jnp.float32)[None, :]

    nr = n // BR
    nc = k // BC

    out = pl.pallas_call(
        _vq_kernel,
        grid=(nr, nc + 1),
        in_specs=[
            pl.BlockSpec((BR, d), lambda i, j: (i, 0)),
            pl.BlockSpec((BC, d), lambda i, j: (jnp.minimum(j, nc - 1), 0)),
            pl.BlockSpec((BR, 1), lambda i, j: (i, 0)),
            pl.BlockSpec((1, BC), lambda i, j: (0, jnp.maximum(j - 1, 0))),
            pl.BlockSpec((1, BC), lambda i, j: (0, jnp.maximum(j - 1, 0))),
        ],
        out_specs=pl.BlockSpec((BR, 1), lambda i, j: (i, 0)),
        out_shape=jax.ShapeDtypeStruct((n, 1), jnp.int32),
        scratch_shapes=[
            pltpu.VMEM((2 * BR, BC), jnp.float32),
            pltpu.VMEM((BR, LANES), jnp.float32),
            pltpu.VMEM((BR, LANES), jnp.float32),
            pltpu.VMEM((BR, 1), jnp.float32),
            pltpu.VMEM((BR, 1), jnp.float32),
        ],
    )(xb, vb, x2, v2, colf)

    return out.reshape(shape).astype(jnp.int64)


# speedup vs baseline: 1.3371x; 1.3371x over previous
import functools

import jax
import jax.numpy as jnp
from jax.experimental import pallas as pl
from jax.experimental.pallas import tpu as pltpu

BR = 512
BC = 2048
LANES = 128


def _vq_kernel(x_ref, v_ref, x2_ref, v2_ref, col_ref, out_ref,
               sv, sc, val_lo, idx_lo):
    j = pl.program_id(1)
    ncols = pl.num_programs(1)
    half = ncols // 2

    m = jax.lax.dot_general(
        x_ref[...], v_ref[...],
        dimension_numbers=(((1,), (1,)), ((), ())),
        preferred_element_type=jnp.float32,
    )
    s = x2_ref[...] + v2_ref[...]
    d2 = jnp.float32(-2.0) * m + s
    dist = jnp.sqrt(jnp.maximum(d2, 0.0))
    col = col_ref[...]

    @pl.when((j == 0) | (j == half))
    def _reset():
        sv[...] = jnp.full((BR, LANES), jnp.inf, jnp.float32)
        sc[...] = jnp.zeros((BR, LANES), jnp.float32)

    tv = dist[:, :LANES]
    tc = jnp.broadcast_to(col[:, :LANES], (BR, LANES))
    for c in range(1, BC // LANES):
        ch = dist[:, c * LANES:(c + 1) * LANES]
        cc = col[:, c * LANES:(c + 1) * LANES]
        better = ch < tv
        tv = jnp.where(better, ch, tv)
        tc = jnp.where(better, cc, tc)
    bet = tv < sv[...]
    sv[...] = jnp.where(bet, tv, sv[...])
    sc[...] = jnp.where(bet, tc, sc[...])

    def _resolve():
        vst = sv[...]
        rv = jnp.min(vst, axis=1, keepdims=True)
        ri = jnp.min(jnp.where(vst == rv, sc[...], jnp.inf), axis=1,
                     keepdims=True)
        return rv, ri

    @pl.when(j == half - 1)
    def _save_lo():
        rv, ri = _resolve()
        val_lo[...] = rv
        idx_lo[...] = ri

    @pl.when(j == ncols - 1)
    def _emit():
        rv, ri = _resolve()
        lo_rounded = val_lo[...].astype(jnp.bfloat16).astype(jnp.float32)
        take_hi = rv < lo_rounded
        best = jnp.where(take_hi, ri, idx_lo[...])
        out_ref[...] = best.astype(jnp.int32)


@functools.partial(jax.jit, static_argnames=())
def kernel(x, vectors):
    shape = x.shape[:-1]
    d = x.shape[-1]
    xf = x.reshape(-1, d)
    n = xf.shape[0]
    k = vectors.shape[0]

    x2 = jnp.sum(xf * xf, axis=-1, keepdims=True)
    v2 = jnp.sum(vectors * vectors, axis=-1)[None, :]

    xb = xf.astype(jnp.bfloat16)
    vb = vectors.astype(jnp.bfloat16)
    colf = jnp.arange(k, dtype=jnp.float32)[None, :]

    nr = n // BR
    nc = k // BC

    out = pl.pallas_call(
        _vq_kernel,
        grid=(nr, nc),
        in_specs=[
            pl.BlockSpec((BR, d), lambda i, j: (i, 0)),
            pl.BlockSpec((BC, d), lambda i, j: (j, 0)),
            pl.BlockSpec((BR, 1), lambda i, j: (i, 0)),
            pl.BlockSpec((1, BC), lambda i, j: (0, j)),
            pl.BlockSpec((1, BC), lambda i, j: (0, j)),
        ],
        out_specs=pl.BlockSpec((BR, 1), lambda i, j: (i, 0)),
        out_shape=jax.ShapeDtypeStruct((n, 1), jnp.int32),
        scratch_shapes=[
            pltpu.VMEM((BR, LANES), jnp.float32),
            pltpu.VMEM((BR, LANES), jnp.float32),
            pltpu.VMEM((BR, 1), jnp.float32),
            pltpu.VMEM((BR, 1), jnp.float32),
        ],
    )(xb, vb, x2, v2, colf)

    return out.reshape(shape).astype(jnp.int64)


# speedup vs baseline: 1.3809x; 1.0327x over previous
import functools

import jax
import jax.numpy as jnp
from jax.experimental import pallas as pl
from jax.experimental.pallas import tpu as pltpu

BR = 512
BC = 4096
LANES = 128


def _vq_kernel(x_ref, v_ref, x2_ref, v2_ref, col_ref, out_ref,
               sv, sc, val_lo, idx_lo):
    j = pl.program_id(1)
    ncols = pl.num_programs(1)
    half = ncols // 2

    m = jax.lax.dot_general(
        x_ref[...], v_ref[...],
        dimension_numbers=(((1,), (1,)), ((), ())),
        preferred_element_type=jnp.float32,
    )
    s = x2_ref[...] + v2_ref[...]
    d2 = jnp.float32(-2.0) * m + s
    dist = jnp.sqrt(jnp.maximum(d2, 0.0))
    col = col_ref[...]

    @pl.when((j == 0) | (j == half))
    def _reset():
        sv[...] = jnp.full((BR, LANES), jnp.inf, jnp.float32)
        sc[...] = jnp.zeros((BR, LANES), jnp.float32)

    tv = dist[:, :LANES]
    tc = jnp.broadcast_to(col[:, :LANES], (BR, LANES))
    for c in range(1, BC // LANES):
        ch = dist[:, c * LANES:(c + 1) * LANES]
        cc = col[:, c * LANES:(c + 1) * LANES]
        better = ch < tv
        tv = jnp.where(better, ch, tv)
        tc = jnp.where(better, cc, tc)
    bet = tv < sv[...]
    sv[...] = jnp.where(bet, tv, sv[...])
    sc[...] = jnp.where(bet, tc, sc[...])

    def _resolve():
        vst = sv[...]
        rv = jnp.min(vst, axis=1, keepdims=True)
        ri = jnp.min(jnp.where(vst == rv, sc[...], jnp.inf), axis=1,
                     keepdims=True)
        return rv, ri

    @pl.when(j == half - 1)
    def _save_lo():
        rv, ri = _resolve()
        val_lo[...] = rv
        idx_lo[...] = ri

    @pl.when(j == ncols - 1)
    def _emit():
        rv, ri = _resolve()
        lo_rounded = val_lo[...].astype(jnp.bfloat16).astype(jnp.float32)
        take_hi = rv < lo_rounded
        best = jnp.where(take_hi, ri, idx_lo[...])
        out_ref[...] = best.astype(jnp.int32)


@functools.partial(jax.jit, static_argnames=())
def kernel(x, vectors):
    shape = x.shape[:-1]
    d = x.shape[-1]
    xf = x.reshape(-1, d)
    n = xf.shape[0]
    k = vectors.shape[0]

    x2 = jnp.sum(xf * xf, axis=-1, keepdims=True)
    v2 = jnp.sum(vectors * vectors, axis=-1)[None, :]

    xb = xf.astype(jnp.bfloat16)
    vb = vectors.astype(jnp.bfloat16)
    colf = jnp.arange(k, dtype=jnp.float32)[None, :]

    nr = n // BR
    nc = k // BC

    out = pl.pallas_call(
        _vq_kernel,
        grid=(nr, nc),
        in_specs=[
            pl.BlockSpec((BR, d), lambda i, j: (i, 0)),
            pl.BlockSpec((BC, d), lambda i, j: (j, 0)),
            pl.BlockSpec((BR, 1), lambda i, j: (i, 0)),
            pl.BlockSpec((1, BC), lambda i, j: (0, j)),
            pl.BlockSpec((1, BC), lambda i, j: (0, j)),
        ],
        out_specs=pl.BlockSpec((BR, 1), lambda i, j: (i, 0)),
        out_shape=jax.ShapeDtypeStruct((n, 1), jnp.int32),
        scratch_shapes=[
            pltpu.VMEM((BR, LANES), jnp.float32),
            pltpu.VMEM((BR, LANES), jnp.float32),
            pltpu.VMEM((BR, 1), jnp.float32),
            pltpu.VMEM((BR, 1), jnp.float32),
        ],
    )(xb, vb, x2, v2, colf)

    return out.reshape(shape).astype(jnp.int64)


# speedup vs baseline: 1.4012x; 1.0147x over previous
import functools

import jax
import jax.numpy as jnp
from jax.experimental import pallas as pl
from jax.experimental.pallas import tpu as pltpu

BR = 512
BC = 4096
LANES = 128


def _vq_kernel(x_ref, v_ref, x2_ref, v2_ref, col_ref, out_ref,
               val_lo, idx_lo):
    j = pl.program_id(1)
    ncols = pl.num_programs(1)

    m = jax.lax.dot_general(
        x_ref[...], v_ref[...],
        dimension_numbers=(((1,), (1,)), ((), ())),
        preferred_element_type=jnp.float32,
    )
    s = x2_ref[...] + v2_ref[...]
    d2 = jnp.float32(-2.0) * m + s
    dist = jnp.sqrt(jnp.maximum(d2, 0.0))
    col = col_ref[...]

    tv = dist[:, :LANES]
    tc = jnp.broadcast_to(col[:, :LANES], (BR, LANES))
    for c in range(1, BC // LANES):
        ch = dist[:, c * LANES:(c + 1) * LANES]
        cc = col[:, c * LANES:(c + 1) * LANES]
        better = ch < tv
        tv = jnp.where(better, ch, tv)
        tc = jnp.where(better, cc, tc)

    rv = jnp.min(tv, axis=1, keepdims=True)
    ri = jnp.min(jnp.where(tv == rv, tc, jnp.inf), axis=1, keepdims=True)

    @pl.when(j == 0)
    def _save_lo():
        val_lo[...] = rv
        idx_lo[...] = ri

    @pl.when(j == ncols - 1)
    def _emit():
        lo_rounded = val_lo[...].astype(jnp.bfloat16).astype(jnp.float32)
        take_hi = rv < lo_rounded
        best = jnp.where(take_hi, ri, idx_lo[...])
        out_ref[...] = best.astype(jnp.int32)


@functools.partial(jax.jit, static_argnames=())
def kernel(x, vectors):
    shape = x.shape[:-1]
    d = x.shape[-1]
    xf = x.reshape(-1, d)
    n = xf.shape[0]
    k = vectors.shape[0]

    x2 = jnp.sum(xf * xf, axis=-1, keepdims=True)
    v2 = jnp.sum(vectors * vectors, axis=-1)[None, :]

    xb = xf.astype(jnp.bfloat16)
    vb = vectors.astype(jnp.bfloat16)
    colf = jnp.arange(k, dtype=jnp.float32)[None, :]

    nr = n // BR
    nc = k // BC

    out = pl.pallas_call(
        _vq_kernel,
        grid=(nr, nc),
        in_specs=[
            pl.BlockSpec((BR, d), lambda i, j: (i, 0)),
            pl.BlockSpec((BC, d), lambda i, j: (j, 0)),
            pl.BlockSpec((BR, 1), lambda i, j: (i, 0)),
            pl.BlockSpec((1, BC), lambda i, j: (0, j)),
            pl.BlockSpec((1, BC), lambda i, j: (0, j)),
        ],
        out_specs=pl.BlockSpec((BR, 1), lambda i, j: (i, 0)),
        out_shape=jax.ShapeDtypeStruct((n, 1), jnp.int32),
        scratch_shapes=[
            pltpu.VMEM((BR, 1), jnp.float32),
            pltpu.VMEM((BR, 1), jnp.float32),
        ],
    )(xb, vb, x2, v2, colf)

    return out.reshape(shape).astype(jnp.int64)


# speedup vs baseline: 1.5596x; 1.1131x over previous
import functools

import jax
import jax.numpy as jnp
from jax.experimental import pallas as pl
from jax.experimental.pallas import tpu as pltpu

BR = 1024
BC = 4096
LANES = 128


def _vq_kernel(x_ref, v_ref, x2_ref, v2_ref, col_ref, out_ref,
               val_lo, idx_lo):
    j = pl.program_id(1)
    ncols = pl.num_programs(1)

    m = jax.lax.dot_general(
        x_ref[...], v_ref[...],
        dimension_numbers=(((1,), (1,)), ((), ())),
        preferred_element_type=jnp.float32,
    )
    s = x2_ref[...] + v2_ref[...]
    d2 = jnp.float32(-2.0) * m + s
    dist = jnp.sqrt(jnp.maximum(d2, 0.0))
    col = col_ref[...]

    tv = dist[:, :LANES]
    tc = jnp.broadcast_to(col[:, :LANES], (BR, LANES))
    for c in range(1, BC // LANES):
        ch = dist[:, c * LANES:(c + 1) * LANES]
        cc = col[:, c * LANES:(c + 1) * LANES]
        better = ch < tv
        tv = jnp.where(better, ch, tv)
        tc = jnp.where(better, cc, tc)

    rv = jnp.min(tv, axis=1, keepdims=True)
    ri = jnp.min(jnp.where(tv == rv, tc, jnp.inf), axis=1, keepdims=True)

    @pl.when(j == 0)
    def _save_lo():
        val_lo[...] = rv
        idx_lo[...] = ri

    @pl.when(j == ncols - 1)
    def _emit():
        lo_rounded = val_lo[...].astype(jnp.bfloat16).astype(jnp.float32)
        take_hi = rv < lo_rounded
        best = jnp.where(take_hi, ri, idx_lo[...])
        out_ref[...] = best.astype(jnp.int32)


@functools.partial(jax.jit, static_argnames=())
def kernel(x, vectors):
    shape = x.shape[:-1]
    d = x.shape[-1]
    xf = x.reshape(-1, d)
    n = xf.shape[0]
    k = vectors.shape[0]

    x2 = jnp.sum(xf * xf, axis=-1, keepdims=True)
    v2 = jnp.sum(vectors * vectors, axis=-1)[None, :]

    xb = xf.astype(jnp.bfloat16)
    vb = vectors.astype(jnp.bfloat16)
    colf = jnp.arange(k, dtype=jnp.float32)[None, :]

    nr = n // BR
    nc = k // BC

    out = pl.pallas_call(
        _vq_kernel,
        grid=(nr, nc),
        in_specs=[
            pl.BlockSpec((BR, d), lambda i, j: (i, 0)),
            pl.BlockSpec((BC, d), lambda i, j: (j, 0)),
            pl.BlockSpec((BR, 1), lambda i, j: (i, 0)),
            pl.BlockSpec((1, BC), lambda i, j: (0, j)),
            pl.BlockSpec((1, BC), lambda i, j: (0, j)),
        ],
        out_specs=pl.BlockSpec((BR, 1), lambda i, j: (i, 0)),
        out_shape=jax.ShapeDtypeStruct((n, 1), jnp.int32),
        scratch_shapes=[
            pltpu.VMEM((BR, 1), jnp.float32),
            pltpu.VMEM((BR, 1), jnp.float32),
        ],
    )(xb, vb, x2, v2, colf)

    return out.reshape(shape).astype(jnp.int64)
